# baseline (speedup 1.0000x reference)
import math
from functools import partial

import jax
import jax.numpy as jnp
from jax.experimental import pallas as pl
from jax.experimental.pallas import tpu as pltpu


def _gelu_tanh(x):
    u = 0.7978845608028654 * (x + 0.044715 * (x * x * x))
    return 0.5 * x * (1.0 + jnp.tanh(u))


def _encoder_kernel(x_ref, win_hbm, wout_hbm, w1_hbm, w2_hbm,
                    bqkv_r, bout_r, b1_r, b2_r, g1_r, be1_r, g2_r, be2_r,
                    o_ref,
                    wqkv_b, wo_b, w1_b, w2_b, qkv_b, ctx_b, stage_a, stage_b,
                    sem_a, sem_b,
                    *, nhead, bt, seq, scale):
    f32 = jnp.float32
    bf16 = jnp.bfloat16
    S, BT, D = x_ref.shape
    N = BT * S
    F = w1_b.shape[0]
    hd = D // nhead
    half = F // 2
    w2c = D // 4

    def start_a(hbm, r0, slot):
        pltpu.make_async_copy(hbm.at[pl.ds(r0, D), :], stage_a.at[slot],
                              sem_a.at[slot]).start()

    def take_a(hbm, slot, dst, r0):
        pltpu.make_async_copy(hbm.at[pl.ds(0, D), :], stage_a.at[slot],
                              sem_a.at[slot]).wait()
        dst[pl.ds(r0, D), :] = stage_a[slot].astype(bf16)

    def start_b(r0, slot):
        pltpu.make_async_copy(w2_hbm.at[pl.ds(r0, w2c), :], stage_b.at[slot],
                              sem_b.at[slot]).start()

    def take_b(slot, r0):
        pltpu.make_async_copy(w2_hbm.at[pl.ds(0, w2c), :], stage_b.at[slot],
                              sem_b.at[slot]).wait()
        w2_b[pl.ds(r0, w2c), :] = stage_b[slot].astype(bf16)

    start_a(win_hbm, 0, 0)
    start_a(win_hbm, D, 1)

    xT = jnp.concatenate([x_ref[:, b, :].T for b in range(bt)], axis=1)

    b_qkv = bqkv_r[...].reshape(3 * D, 1)
    b_out = bout_r[...].reshape(D, 1)
    b1 = b1_r[...].reshape(F, 1)
    b2 = b2_r[...].reshape(D, 1)
    g1 = g1_r[...].reshape(D, 1)
    be1 = be1_r[...].reshape(D, 1)
    g2 = g2_r[...].reshape(D, 1)
    be2 = be2_r[...].reshape(D, 1)

    def ln(z, g, b):
        mu = jnp.mean(z, axis=0, keepdims=True)
        zc = z - mu
        var = jnp.mean(zc * zc, axis=0, keepdims=True)
        return zc * jax.lax.rsqrt(var + 1e-5) * g + b

    y = ln(xT, g1, be1).astype(bf16)

    take_a(win_hbm, 0, wqkv_b, 0)
    start_a(win_hbm, 2 * D, 0)
    take_a(win_hbm, 1, wqkv_b, D)
    start_a(wout_hbm, 0, 1)
    take_a(win_hbm, 0, wqkv_b, 2 * D)
    start_a(w1_hbm, 0, 0)

    qkv = (jnp.dot(wqkv_b[...], y, preferred_element_type=f32)
           + b_qkv).astype(bf16)

    take_a(wout_hbm, 1, wo_b, 0)
    start_a(w1_hbm, D, 1)

    qkv_b[...] = qkv

    def attend(b, _):
        c0 = b * seq
        sl = qkv_b[:, pl.ds(c0, seq)]
        sl3 = sl.reshape(3 * nhead, hd, seq)
        qb = sl3[0:nhead]
        kb = sl3[nhead:2 * nhead]
        vb = sl3[2 * nhead:3 * nhead]
        s = jnp.einsum('heq,hek->hqk', qb, kb,
                       preferred_element_type=f32)
        s = s * scale
        s = s - jnp.max(s, axis=2, keepdims=True)
        p = jnp.exp(s)
        p = (p * pl.reciprocal(jnp.sum(p, axis=2, keepdims=True),
                               approx=True)).astype(bf16)
        c = jnp.einsum('hek,hqk->heq', vb, p,
                       preferred_element_type=f32)
        ctx_b[:, pl.ds(c0, seq)] = c.reshape(D, seq).astype(bf16)
        return ()

    jax.lax.fori_loop(0, bt, attend, (), unroll=2)

    take_a(w1_hbm, 0, w1_b, 0)
    start_a(w1_hbm, 2 * D, 0)
    take_a(w1_hbm, 1, w1_b, D)
    start_a(w1_hbm, 3 * D, 1)
    take_a(w1_hbm, 0, w1_b, 2 * D)
    start_b(0, 0)
    take_a(w1_hbm, 1, w1_b, 3 * D)
    start_b(w2c, 1)
    ctxT = ctx_b[...]

    attn = jnp.dot(wo_b[...], ctxT, preferred_element_type=f32) + b_out
    x1 = xT + attn
    y2 = ln(x1, g2, be2).astype(bf16)

    take_b(0, 0)
    start_b(2 * w2c, 0)
    take_b(1, w2c)
    start_b(3 * w2c, 1)

    h1a = jnp.dot(w1_b[0:half, :], y2, preferred_element_type=f32) + b1[0:half]
    h1a = _gelu_tanh(h1a).astype(bf16)

    take_b(0, 2 * w2c)
    take_b(1, 3 * w2c)

    h1b = jnp.dot(w1_b[half:F, :], y2, preferred_element_type=f32) + b1[half:F]
    h1b = _gelu_tanh(h1b).astype(bf16)

    out = (x1 + b2
           + jnp.dot(w2_b[:, 0:half], h1a, preferred_element_type=f32)
           + jnp.dot(w2_b[:, half:F], h1b, preferred_element_type=f32))

    for b in range(bt):
        o_ref[:, b, :] = out[:, b * seq:(b + 1) * seq].T


def kernel(src, w_in, b_in, w_out, b_out, w1, b1, w2, b2, g1, be1, g2, be2):
    S, B, D = src.shape
    H = 12
    hd = D // H
    F = w1.shape[0]
    scale = 1.0 / math.sqrt(hd)
    f32, bf16 = jnp.float32, jnp.bfloat16

    BT = 8 if B % 8 == 0 else B
    grid = (B // BT,)

    def _row(v):
        return v.reshape(1, v.shape[0]).astype(f32)

    def _vmem(shape):
        nd = len(shape)
        return pl.BlockSpec(shape, lambda b, _nd=nd: (0,) * _nd)

    hbm = pl.BlockSpec(memory_space=pl.ANY)

    out = pl.pallas_call(
        partial(_encoder_kernel, nhead=H, bt=BT, seq=S, scale=scale),
        out_shape=jax.ShapeDtypeStruct((S, B, D), f32),
        grid=grid,
        in_specs=[pl.BlockSpec((S, BT, D), lambda b: (0, b, 0)),
                  hbm, hbm, hbm, hbm,
                  _vmem((1, 3 * D)), _vmem((1, D)), _vmem((1, F)),
                  _vmem((1, D)), _vmem((1, D)), _vmem((1, D)),
                  _vmem((1, D)), _vmem((1, D))],
        out_specs=pl.BlockSpec((S, BT, D), lambda b: (0, b, 0)),
        scratch_shapes=[pltpu.VMEM((3 * D, D), bf16),
                        pltpu.VMEM((D, D), bf16),
                        pltpu.VMEM((F, D), bf16),
                        pltpu.VMEM((D, F), bf16),
                        pltpu.VMEM((3 * D, BT * S), bf16),
                        pltpu.VMEM((D, BT * S), bf16),
                        pltpu.VMEM((2, D, D), f32),
                        pltpu.VMEM((2, D // 4, F), f32),
                        pltpu.SemaphoreType.DMA((2,)),
                        pltpu.SemaphoreType.DMA((2,))],
        compiler_params=pltpu.CompilerParams(
            dimension_semantics=("parallel",),
            vmem_limit_bytes=64 * 1024 * 1024,
        ),
    )(src.astype(f32), w_in.astype(f32), w_out.astype(f32),
      w1.astype(f32), w2.astype(f32),
      _row(b_in), _row(b_out), _row(b1), _row(b2),
      _row(g1), _row(be1), _row(g2), _row(be2))

    return out


# speedup vs baseline: 1.1157x; 1.1157x over previous
import math
from functools import partial

import jax
import jax.numpy as jnp
from jax.experimental import pallas as pl
from jax.experimental.pallas import tpu as pltpu


def _gelu_tanh(x):
    u = 0.7978845608028654 * (x + 0.044715 * (x * x * x))
    return 0.5 * x * (1.0 + jnp.tanh(u))


def _encoder_kernel(x_ref, win_hbm, wout_hbm, w1_hbm, w2_hbm,
                    bqkv_r, bout_r, b1_r, b2_r, g1_r, be1_r, g2_r, be2_r,
                    o_ref,
                    wqkv_b, wo_b, w1_b, w2_b, qkv_b, ctx_b, stage_a, stage_b,
                    sem_a, sem_b,
                    *, nhead, bt, seq, scale):
    f32 = jnp.float32
    bf16 = jnp.bfloat16
    S, BT, D = x_ref.shape
    N = BT * S
    F = w1_b.shape[0]
    hd = D // nhead
    half = F // 2
    w2c = D // 4

    def start_a(hbm, r0, slot):
        pltpu.make_async_copy(hbm.at[pl.ds(r0, D), :], stage_a.at[slot],
                              sem_a.at[slot]).start()

    def take_a(hbm, slot, dst, r0):
        pltpu.make_async_copy(hbm.at[pl.ds(0, D), :], stage_a.at[slot],
                              sem_a.at[slot]).wait()
        dst[pl.ds(r0, D), :] = stage_a[slot].astype(bf16)

    def start_b(r0, slot):
        pltpu.make_async_copy(w2_hbm.at[pl.ds(r0, w2c), :], stage_b.at[slot],
                              sem_b.at[slot]).start(priority=1)

    def take_b(slot, r0):
        pltpu.make_async_copy(w2_hbm.at[pl.ds(0, w2c), :], stage_b.at[slot],
                              sem_b.at[slot]).wait()
        w2_b[pl.ds(r0, w2c), :] = stage_b[slot].astype(bf16)

    start_a(win_hbm, 0, 0)
    start_a(win_hbm, D, 1)
    start_a(win_hbm, 2 * D, 2)
    start_b(0, 0)
    start_b(w2c, 1)

    xT = jnp.concatenate([x_ref[:, b, :].T for b in range(bt)], axis=1)

    b_qkv = bqkv_r[...].reshape(3 * D, 1)
    b_out = bout_r[...].reshape(D, 1)
    b1 = b1_r[...].reshape(F, 1)
    b2 = b2_r[...].reshape(D, 1)
    g1 = g1_r[...].reshape(D, 1)
    be1 = be1_r[...].reshape(D, 1)
    g2 = g2_r[...].reshape(D, 1)
    be2 = be2_r[...].reshape(D, 1)

    def ln(z, g, b):
        mu = jnp.mean(z, axis=0, keepdims=True)
        zc = z - mu
        var = jnp.mean(zc * zc, axis=0, keepdims=True)
        return zc * jax.lax.rsqrt(var + 1e-5) * g + b

    y = ln(xT, g1, be1).astype(bf16)

    take_a(win_hbm, 0, wqkv_b, 0)
    start_a(wout_hbm, 0, 0)
    take_a(win_hbm, 1, wqkv_b, D)
    start_a(w1_hbm, 0, 1)
    take_a(win_hbm, 2, wqkv_b, 2 * D)
    start_a(w1_hbm, D, 2)

    qkv = (jnp.dot(wqkv_b[...], y, preferred_element_type=f32)
           + b_qkv).astype(bf16)

    take_a(wout_hbm, 0, wo_b, 0)
    start_a(w1_hbm, 2 * D, 0)
    take_a(w1_hbm, 1, w1_b, 0)
    start_a(w1_hbm, 3 * D, 1)

    qkv_b[...] = qkv

    def attend(b, _):
        c0 = b * seq
        sl = qkv_b[:, pl.ds(c0, seq)]
        sl3 = sl.reshape(3 * nhead, hd, seq)
        qb = sl3[0:nhead]
        kb = sl3[nhead:2 * nhead]
        vb = sl3[2 * nhead:3 * nhead]
        s = jnp.einsum('heq,hek->hqk', qb, kb,
                       preferred_element_type=f32)
        s = s * scale
        s = s - jnp.max(s, axis=2, keepdims=True)
        p = jnp.exp(s)
        p = (p * pl.reciprocal(jnp.sum(p, axis=2, keepdims=True),
                               approx=True)).astype(bf16)
        c = jnp.einsum('hek,hqk->heq', vb, p,
                       preferred_element_type=f32)
        ctx_b[:, pl.ds(c0, seq)] = c.reshape(D, seq).astype(bf16)
        return ()

    jax.lax.fori_loop(0, bt, attend, (), unroll=2)

    take_a(w1_hbm, 2, w1_b, D)
    take_a(w1_hbm, 0, w1_b, 2 * D)
    take_a(w1_hbm, 1, w1_b, 3 * D)
    take_b(0, 0)
    start_b(2 * w2c, 0)
    take_b(1, w2c)
    start_b(3 * w2c, 1)
    ctxT = ctx_b[...]

    attn = jnp.dot(wo_b[...], ctxT, preferred_element_type=f32) + b_out
    x1 = xT + attn
    y2 = ln(x1, g2, be2).astype(bf16)

    h1a = jnp.dot(w1_b[0:half, :], y2, preferred_element_type=f32) + b1[0:half]
    h1a = _gelu_tanh(h1a).astype(bf16)

    take_b(0, 2 * w2c)
    take_b(1, 3 * w2c)

    h1b = jnp.dot(w1_b[half:F, :], y2, preferred_element_type=f32) + b1[half:F]
    h1b = _gelu_tanh(h1b).astype(bf16)

    out = (x1 + b2
           + jnp.dot(w2_b[:, 0:half], h1a, preferred_element_type=f32)
           + jnp.dot(w2_b[:, half:F], h1b, preferred_element_type=f32))

    for b in range(bt):
        o_ref[:, b, :] = out[:, b * seq:(b + 1) * seq].T


def kernel(src, w_in, b_in, w_out, b_out, w1, b1, w2, b2, g1, be1, g2, be2):
    S, B, D = src.shape
    H = 12
    hd = D // H
    F = w1.shape[0]
    scale = 1.0 / math.sqrt(hd)
    f32, bf16 = jnp.float32, jnp.bfloat16

    BT = 8 if B % 8 == 0 else B
    grid = (B // BT,)

    def _row(v):
        return v.reshape(1, v.shape[0]).astype(f32)

    def _vmem(shape):
        nd = len(shape)
        return pl.BlockSpec(shape, lambda b, _nd=nd: (0,) * _nd)

    hbm = pl.BlockSpec(memory_space=pl.ANY)

    out = pl.pallas_call(
        partial(_encoder_kernel, nhead=H, bt=BT, seq=S, scale=scale),
        out_shape=jax.ShapeDtypeStruct((S, B, D), f32),
        grid=grid,
        in_specs=[pl.BlockSpec((S, BT, D), lambda b: (0, b, 0)),
                  hbm, hbm, hbm, hbm,
                  _vmem((1, 3 * D)), _vmem((1, D)), _vmem((1, F)),
                  _vmem((1, D)), _vmem((1, D)), _vmem((1, D)),
                  _vmem((1, D)), _vmem((1, D))],
        out_specs=pl.BlockSpec((S, BT, D), lambda b: (0, b, 0)),
        scratch_shapes=[pltpu.VMEM((3 * D, D), bf16),
                        pltpu.VMEM((D, D), bf16),
                        pltpu.VMEM((F, D), bf16),
                        pltpu.VMEM((D, F), bf16),
                        pltpu.VMEM((3 * D, BT * S), bf16),
                        pltpu.VMEM((D, BT * S), bf16),
                        pltpu.VMEM((3, D, D), f32),
                        pltpu.VMEM((2, D // 4, F), f32),
                        pltpu.SemaphoreType.DMA((3,)),
                        pltpu.SemaphoreType.DMA((2,))],
        compiler_params=pltpu.CompilerParams(
            dimension_semantics=("parallel",),
            vmem_limit_bytes=64 * 1024 * 1024,
        ),
    )(src.astype(f32), w_in.astype(f32), w_out.astype(f32),
      w1.astype(f32), w2.astype(f32),
      _row(b_in), _row(b_out), _row(b1), _row(b2),
      _row(g1), _row(be1), _row(g2), _row(be2))

    return out


# speedup vs baseline: 1.1165x; 1.0008x over previous
import math
from functools import partial

import jax
import jax.numpy as jnp
from jax.experimental import pallas as pl
from jax.experimental.pallas import tpu as pltpu


def _gelu_tanh(x):
    u = 0.7978845608028654 * (x + 0.044715 * (x * x * x))
    return 0.5 * x * (1.0 + jnp.tanh(u))


def _encoder_kernel(x_ref, win_hbm, wout_hbm, w1_hbm, w2_hbm,
                    bqkv_r, bout_r, b1_r, b2_r, g1_r, be1_r, g2_r, be2_r,
                    o_ref,
                    wqkv_b, wo_b, w1_b, w2_b, qkv_b, ctx_b, stage_a, stage_b,
                    sem_a, sem_b,
                    *, nhead, bt, seq, scale):
    f32 = jnp.float32
    bf16 = jnp.bfloat16
    S, BT, D = x_ref.shape
    N = BT * S
    F = w1_b.shape[0]
    hd = D // nhead
    half = F // 2
    w2c = D // 4

    def start_a(hbm, r0, slot, prio=0):
        pltpu.make_async_copy(hbm.at[pl.ds(r0, D), :], stage_a.at[slot],
                              sem_a.at[slot]).start(priority=prio)

    def take_a(hbm, slot, dst, r0):
        pltpu.make_async_copy(hbm.at[pl.ds(0, D), :], stage_a.at[slot],
                              sem_a.at[slot]).wait()
        dst[pl.ds(r0, D), :] = stage_a[slot].astype(bf16)

    def start_b(r0, slot):
        pltpu.make_async_copy(w2_hbm.at[pl.ds(r0, w2c), :], stage_b.at[slot],
                              sem_b.at[slot]).start(priority=1)

    def take_b(slot, r0):
        pltpu.make_async_copy(w2_hbm.at[pl.ds(0, w2c), :], stage_b.at[slot],
                              sem_b.at[slot]).wait()
        w2_b[pl.ds(r0, w2c), :] = stage_b[slot].astype(bf16)

    start_a(win_hbm, 0, 0)
    start_a(win_hbm, D, 1)
    start_a(win_hbm, 2 * D, 2)
    start_b(0, 0)
    start_b(w2c, 1)

    xT = jnp.concatenate([x_ref[:, b, :].T for b in range(bt)], axis=1)

    b_qkv = bqkv_r[...].reshape(3 * D, 1)
    b_out = bout_r[...].reshape(D, 1)
    b1 = b1_r[...].reshape(F, 1)
    b2 = b2_r[...].reshape(D, 1)
    g1 = g1_r[...].reshape(D, 1)
    be1 = be1_r[...].reshape(D, 1)
    g2 = g2_r[...].reshape(D, 1)
    be2 = be2_r[...].reshape(D, 1)

    def ln(z, g, b):
        mu = jnp.mean(z, axis=0, keepdims=True)
        zc = z - mu
        var = jnp.mean(zc * zc, axis=0, keepdims=True)
        return zc * jax.lax.rsqrt(var + 1e-5) * g + b

    y = ln(xT, g1, be1).astype(bf16)

    take_a(win_hbm, 0, wqkv_b, 0)
    start_a(wout_hbm, 0, 0)
    take_a(win_hbm, 1, wqkv_b, D)
    start_a(w1_hbm, 0, 1, prio=1)
    take_a(win_hbm, 2, wqkv_b, 2 * D)
    start_a(w1_hbm, D, 2, prio=1)

    qkv = (jnp.dot(wqkv_b[...], y, preferred_element_type=f32)
           + b_qkv).astype(bf16)

    take_a(wout_hbm, 0, wo_b, 0)
    start_a(w1_hbm, 2 * D, 0)
    take_a(w1_hbm, 1, w1_b, 0)
    start_a(w1_hbm, 3 * D, 1)

    qkv_b[...] = qkv

    def attend(b, _):
        c0 = b * seq
        sl = qkv_b[:, pl.ds(c0, seq)]
        sl3 = sl.reshape(3 * nhead, hd, seq)
        qb = sl3[0:nhead]
        kb = sl3[nhead:2 * nhead]
        vb = sl3[2 * nhead:3 * nhead]
        s = jnp.einsum('heq,hek->hqk', qb, kb,
                       preferred_element_type=f32)
        s = s * scale
        s = s - jnp.max(s, axis=2, keepdims=True)
        p = jnp.exp(s)
        p = (p * pl.reciprocal(jnp.sum(p, axis=2, keepdims=True),
                               approx=True)).astype(bf16)
        c = jnp.einsum('hek,hqk->heq', vb, p,
                       preferred_element_type=f32)
        ctx_b[:, pl.ds(c0, seq)] = c.reshape(D, seq).astype(bf16)
        return ()

    jax.lax.fori_loop(0, bt, attend, (), unroll=2)

    take_a(w1_hbm, 2, w1_b, D)
    take_a(w1_hbm, 0, w1_b, 2 * D)
    take_a(w1_hbm, 1, w1_b, 3 * D)
    take_b(0, 0)
    start_b(2 * w2c, 0)
    take_b(1, w2c)
    start_b(3 * w2c, 1)
    ctxT = ctx_b[...]

    attn = jnp.dot(wo_b[...], ctxT, preferred_element_type=f32) + b_out
    x1 = xT + attn
    y2 = ln(x1, g2, be2).astype(bf16)

    h1a = jnp.dot(w1_b[0:half, :], y2, preferred_element_type=f32) + b1[0:half]
    h1a = _gelu_tanh(h1a).astype(bf16)

    take_b(0, 2 * w2c)
    take_b(1, 3 * w2c)

    h1b = jnp.dot(w1_b[half:F, :], y2, preferred_element_type=f32) + b1[half:F]
    h1b = _gelu_tanh(h1b).astype(bf16)

    out = (x1 + b2
           + jnp.dot(w2_b[:, 0:half], h1a, preferred_element_type=f32)
           + jnp.dot(w2_b[:, half:F], h1b, preferred_element_type=f32))

    for b in range(bt):
        o_ref[:, b, :] = out[:, b * seq:(b + 1) * seq].T


def kernel(src, w_in, b_in, w_out, b_out, w1, b1, w2, b2, g1, be1, g2, be2):
    S, B, D = src.shape
    H = 12
    hd = D // H
    F = w1.shape[0]
    scale = 1.0 / math.sqrt(hd)
    f32, bf16 = jnp.float32, jnp.bfloat16

    BT = 8 if B % 8 == 0 else B
    grid = (B // BT,)

    def _row(v):
        return v.reshape(1, v.shape[0]).astype(f32)

    def _vmem(shape):
        nd = len(shape)
        return pl.BlockSpec(shape, lambda b, _nd=nd: (0,) * _nd)

    hbm = pl.BlockSpec(memory_space=pl.ANY)

    out = pl.pallas_call(
        partial(_encoder_kernel, nhead=H, bt=BT, seq=S, scale=scale),
        out_shape=jax.ShapeDtypeStruct((S, B, D), f32),
        grid=grid,
        in_specs=[pl.BlockSpec((S, BT, D), lambda b: (0, b, 0)),
                  hbm, hbm, hbm, hbm,
                  _vmem((1, 3 * D)), _vmem((1, D)), _vmem((1, F)),
                  _vmem((1, D)), _vmem((1, D)), _vmem((1, D)),
                  _vmem((1, D)), _vmem((1, D))],
        out_specs=pl.BlockSpec((S, BT, D), lambda b: (0, b, 0)),
        scratch_shapes=[pltpu.VMEM((3 * D, D), bf16),
                        pltpu.VMEM((D, D), bf16),
                        pltpu.VMEM((F, D), bf16),
                        pltpu.VMEM((D, F), bf16),
                        pltpu.VMEM((3 * D, BT * S), bf16),
                        pltpu.VMEM((D, BT * S), bf16),
                        pltpu.VMEM((3, D, D), f32),
                        pltpu.VMEM((2, D // 4, F), f32),
                        pltpu.SemaphoreType.DMA((3,)),
                        pltpu.SemaphoreType.DMA((2,))],
        compiler_params=pltpu.CompilerParams(
            dimension_semantics=("parallel",),
            vmem_limit_bytes=64 * 1024 * 1024,
        ),
    )(src.astype(f32), w_in.astype(f32), w_out.astype(f32),
      w1.astype(f32), w2.astype(f32),
      _row(b_in), _row(b_out), _row(b1), _row(b2),
      _row(g1), _row(be1), _row(g2), _row(be2))

    return out
